# baseline (speedup 1.0000x reference)
import functools

import jax
import jax.numpy as jnp
from jax import lax
from jax.experimental import pallas as pl
from jax.experimental.pallas import tpu as pltpu
from jax.experimental.pallas import tpu_sc as plsc

N_NODES = 100000
N_EDGES = 1600000
EMB = 32
HID = 32
NEG_SLOPE = 0.2

NODE_BLK = 1024

NC = 2
NS = 16
NW = NC * NS

_SC_MESH = dict(core_axis_name="c", subcore_axis_name="s")
_SC_PARAMS = pltpu.CompilerParams(use_tc_tiling_on_sc=False)
_SC_PARAMS_NL = pltpu.CompilerParams(use_tc_tiling_on_sc=False,
                                     needs_layout_passes=False)


def _sc_gather_h_body(ht_hbm, nf_hbm, h_hbm, idx_v, rows_v, sem):
    w = lax.axis_index("s") * NC + lax.axis_index("c")
    nblk = 125
    for k in range(4):
        b = w + k * NW
        @pl.when(b < nblk)
        def _():
            off = b * 800
            pltpu.sync_copy(nf_hbm.at[pl.ds(off, 800)], idx_v)
            pltpu.async_copy(ht_hbm.at[idx_v], rows_v, sem).wait()
            pltpu.sync_copy(rows_v, h_hbm.at[pl.ds(off, 800)])


def _sc_gather_eout_body(lnet_hbm, ef_hbm, eout_hbm, idx_v, rows_a, rows_b, sem):
    w = lax.axis_index("s") * NC + lax.axis_index("c")
    base = w * 50000
    bufs = (rows_a, rows_b)

    pltpu.sync_copy(ef_hbm.at[pl.ds(base, 400)], idx_v.at[0])
    pltpu.async_copy(lnet_hbm.at[idx_v.at[0]], rows_a, sem).wait()

    def step(k, _):
        @pl.when(k + 1 < 125)
        def _():
            pltpu.sync_copy(ef_hbm.at[pl.ds(base + (k + 1) * 400, 400)],
                            idx_v.at[(k + 1) % 2])
        for p in range(2):
            @pl.when(lax.rem(k, 2) == p)
            def _():
                @pl.when(k + 1 < 125)
                def _():
                    cp = pltpu.async_copy(lnet_hbm.at[idx_v.at[(p + 1) % 2]],
                                          bufs[(p + 1) % 2], sem)
                    pltpu.sync_copy(bufs[p], eout_hbm.at[pl.ds(base + k * 400, 400)])
                    cp.wait()
                @pl.when(k + 1 >= 125)
                def _():
                    pltpu.sync_copy(bufs[p], eout_hbm.at[pl.ds(base + k * 400, 400)])
        return 0

    lax.fori_loop(0, 125, step, 0)


def _sc_gather_h(ht, node_features):
    return pl.kernel(
        _sc_gather_h_body,
        out_type=jax.ShapeDtypeStruct((N_NODES, HID), jnp.float32),
        mesh=plsc.VectorSubcoreMesh(**_SC_MESH),
        compiler_params=_SC_PARAMS,
        scratch_types=[
            pltpu.VMEM((800,), jnp.int32),
            pltpu.VMEM((800, HID), jnp.float32),
            pltpu.SemaphoreType.DMA,
        ],
    )(ht, node_features)


def _sc_eoutT_body(lnetf_hbm, ef_hbm, eoutT_hbm, tab_v, ef_v, out_v, ssem, wsem):
    w = lax.axis_index("s") * NC + lax.axis_index("c")
    base = w * 50000
    pltpu.sync_copy(lnetf_hbm, tab_v)

    efb = (ef_v.at[0], ef_v.at[1])
    outb = (out_v.at[0], out_v.at[1])

    def stream(k, p):
        pltpu.async_copy(ef_hbm.at[pl.ds(base + k * 400, 400)], efb[p], ssem[p])

    def wait_stream(k, p):
        _wait(ef_hbm.at[pl.ds(base + k * 400, 400)], efb[p], ssem[p])

    def writes(k, p):
        for d in range(HID):
            pltpu.async_copy(outb[p].at[d],
                             eoutT_hbm.at[d].at[pl.ds(base + k * 400, 400)],
                             wsem[p])

    def wait_writes(k, p):
        for d in range(HID):
            _wait(outb[p].at[d],
                  eoutT_hbm.at[d].at[pl.ds(base + k * 400, 400)], wsem[p])

    stream(0, 0)

    def body(k, _):
        for p in range(2):
            @pl.when(lax.rem(k, 2) == p)
            def _():
                @pl.when(k >= 2)
                def _():
                    wait_writes(k, p)

                wait_stream(k, p)

                def chunk(j, _):
                    sl = pl.ds(j * 16, 16)
                    bidx = efb[p][sl] * HID
                    for d in range(HID):
                        outb[p][d, sl] = plsc.load_gather(tab_v, [bidx + d])
                    return 0

                lax.fori_loop(0, 400 // 16, chunk, 0)
                writes(k, p)

                @pl.when(k + 1 < 125)
                def _():
                    stream(k + 1, 1 - p)
        return 0

    lax.fori_loop(0, 125, body, 0)
    wait_writes(0, 0)
    wait_writes(0, 1)


def _sc_gather_eout(ln_et, edge_features):
    eoutT = pl.kernel(
        _sc_eoutT_body,
        out_type=jax.ShapeDtypeStruct((HID, N_EDGES), jnp.float32),
        mesh=plsc.VectorSubcoreMesh(**_SC_MESH),
        compiler_params=_SC_PARAMS_NL,
        scratch_types=[
            pltpu.VMEM((1000 * HID,), jnp.float32),
            pltpu.VMEM((2, 400), jnp.int32),
            pltpu.VMEM((2, HID, 400), jnp.float32),
            (pltpu.SemaphoreType.DMA, pltpu.SemaphoreType.DMA),
            (pltpu.SemaphoreType.DMA, pltpu.SemaphoreType.DMA),
        ],
    )(ln_et.reshape(1000 * HID), edge_features)
    return eoutT.T


def _sij_kernel(h_ref, ai_ref, aj_ref, si_ref, sj_ref):
    h = h_ref[...]
    si_ref[...] = h @ ai_ref[...]
    sj_ref[...] = h @ aj_ref[...]


NPAD = 50048
NPADZ = 100096
DUMMY_ROW = 50040
EB = 160
NGB = N_EDGES // EB


def _wait(src, dst, sem):
    pltpu.make_async_copy(src, dst, sem).wait()


def _sc_ex_body(t_hbm, nb_hbm, ef_hbm, pack_hbm, se_hbm, zz_hbm,
                comb_hbm, z01_hbm,
                pack_v, se_v, t_v, nb_v, ef_v, comb0_v, comb1_v, t2_v, ex2_v,
                z_sp, ssem, wsem, zsem):
    c = lax.axis_index("c")
    s = lax.axis_index("s")
    w = s * NC + c
    pltpu.sync_copy(pack_hbm, pack_v)
    pltpu.sync_copy(se_hbm, se_v)
    pltpu.sync_copy(zz_hbm.at[pl.ds(s * 6256, 6256)], z_sp.at[pl.ds(s * 6256, 6256)])
    plsc.subcore_barrier()

    tb = (t_v.at[0], t_v.at[1])
    nbb = (nb_v.at[0], nb_v.at[1])
    efb = (ef_v.at[0], ef_v.at[1])
    c0b = (comb0_v.at[0], comb0_v.at[1])
    c1b = (comb1_v.at[0], comb1_v.at[1])
    t2b = (t2_v.at[0], t2_v.at[1])
    ex2b = (ex2_v.at[0], ex2_v.at[1])

    nblk = jnp.where(w < 16, 313, 312)

    def row_of(k):
        return w + k * NW

    def streams(k, p):
        off = row_of(k) * EB
        pltpu.async_copy(t_hbm.at[pl.ds(off, EB)], tb[p], ssem[p])
        pltpu.async_copy(nb_hbm.at[pl.ds(off, EB)], nbb[p], ssem[p])
        pltpu.async_copy(ef_hbm.at[pl.ds(off, EB)], efb[p], ssem[p])

    def wait_streams(k, p):
        off = row_of(k) * EB
        _wait(t_hbm.at[pl.ds(off, EB)], tb[p], ssem[p])
        _wait(nb_hbm.at[pl.ds(off, EB)], nbb[p], ssem[p])
        _wait(ef_hbm.at[pl.ds(off, EB)], efb[p], ssem[p])

    def wait_outs(k, p):
        gb = row_of(k)
        _wait(c0b[p], comb_hbm.at[0].at[gb], wsem[p])
        _wait(c1b[p], comb_hbm.at[1].at[gb], wsem[p])
        for b in range(2):
            _wait(ex2b[p].at[b], z_sp.at[t2b[p].at[b]], zsem[p])

    streams(0, 0)

    def body(k, _):
        for p in range(2):
            @pl.when((lax.rem(k, 2) == p) & (k < nblk))
            def _():
                @pl.when(k >= 2)
                def _():
                    wait_outs(k, p)

                wait_streams(k, p)

                def chunk(j, _):
                    t16 = tb[p][pl.ds(j * 16, 16)]
                    n16 = nbb[p][pl.ds(j * 16, 16)]
                    ef16 = efb[p][pl.ds(j * 16, 16)]
                    pk_t = plsc.load_gather(pack_v, [t16])
                    pk_n = plsc.load_gather(pack_v, [n16])
                    se16 = plsc.load_gather(se_v, [ef16])
                    si = plsc.bitcast(lax.shift_left(pk_t, 16), jnp.float32)
                    sj = plsc.bitcast(lax.bitwise_and(pk_n, jnp.int32(-65536)),
                                      jnp.float32)
                    x = si + sj + se16
                    x = jnp.where(x >= 0.0, x, NEG_SLOPE * x)
                    ex = jnp.exp(x)
                    owned0 = t16 < 50000
                    exm0 = jnp.where(owned0, ex, 0.0)
                    exm1 = ex - exm0
                    idx0 = jnp.where(owned0, t16, DUMMY_ROW)
                    idx1 = jnp.where(owned0, DUMMY_ROW, t16 - 50000)
                    r = j // 5
                    sl = pl.ds((j % 5) * 16, 16)
                    c0b[p][0, r, sl] = n16
                    c0b[p][1, r, sl] = idx0
                    c0b[p][2, r, sl] = plsc.bitcast(exm0, jnp.int32)
                    c1b[p][0, r, sl] = n16
                    c1b[p][1, r, sl] = idx1
                    c1b[p][2, r, sl] = plsc.bitcast(exm1, jnp.int32)
                    t2b[p][r, sl] = t16
                    ex2b[p][r, sl] = ex
                    return 0

                lax.fori_loop(0, EB // 16, chunk, 0)
                gb = row_of(k)
                pltpu.async_copy(c0b[p], comb_hbm.at[0].at[gb], wsem[p])
                pltpu.async_copy(c1b[p], comb_hbm.at[1].at[gb], wsem[p])
                for b in range(2):
                    pltpu.async_copy(ex2b[p].at[b], z_sp.at[t2b[p].at[b]],
                                     zsem[p], add=True)

                @pl.when(k + 1 < nblk)
                def _():
                    streams(k + 1, 1 - p)
        return 0

    lax.fori_loop(0, 313, body, 0)
    wait_outs(0, 0)
    wait_outs(0, 1)
    plsc.subcore_barrier()
    pltpu.sync_copy(z_sp.at[pl.ds(s * 6256, 6256)],
                    z01_hbm.at[c].at[pl.ds(s * 6256, 6256)])


def _sc_ex(t_arr, nb_arr, ef_arr, pack, se_t):
    zz = jnp.zeros((NPADZ,), jnp.float32)
    return pl.kernel(
        _sc_ex_body,
        out_type=[
            jax.ShapeDtypeStruct((2, NGB, 3, 2, 80), jnp.int32),
            jax.ShapeDtypeStruct((NC, NPADZ), jnp.float32),
        ],
        mesh=plsc.VectorSubcoreMesh(**_SC_MESH),
        compiler_params=_SC_PARAMS_NL,
        scratch_types=[
            pltpu.VMEM((N_NODES,), jnp.int32),
            pltpu.VMEM((1000,), jnp.float32),
            pltpu.VMEM((2, EB), jnp.int32),
            pltpu.VMEM((2, EB), jnp.int32),
            pltpu.VMEM((2, EB), jnp.int32),
            pltpu.VMEM((2, 3, 2, 80), jnp.int32),
            pltpu.VMEM((2, 3, 2, 80), jnp.int32),
            pltpu.VMEM((2, 2, 80), jnp.int32),
            pltpu.VMEM((2, 2, 80), jnp.float32),
            pltpu.VMEM_SHARED((NPADZ,), jnp.float32),
            (pltpu.SemaphoreType.DMA, pltpu.SemaphoreType.DMA),
            (pltpu.SemaphoreType.DMA, pltpu.SemaphoreType.DMA),
            (pltpu.SemaphoreType.DMA, pltpu.SemaphoreType.DMA),
        ],
    )(t_arr, nb_arr, ef_arr, pack, se_t, zz)


def _sc_acc_body(comb_hbm, h_hbm, zu_hbm, z01_hbm, u_hbm,
                 comb_v, hrows_v, idxc_v, zi_v, z2_v, ub_v,
                 u_sp, gsem, csem, ssem):
    c = lax.axis_index("c")
    s = lax.axis_index("s")

    pltpu.sync_copy(zu_hbm.at[pl.ds(s * 3128, 3128)], u_sp.at[pl.ds(s * 3128, 3128)])
    plsc.subcore_barrier()

    mycomb = comb_hbm.at[c]
    base_blk = s * 625
    combs = (comb_v.at[0], comb_v.at[1], comb_v.at[2])
    hrows = (hrows_v.at[0], hrows_v.at[1], hrows_v.at[2])
    idxcs = (idxc_v.at[0], idxc_v.at[1], idxc_v.at[2])

    def stream(k, r):
        pltpu.async_copy(mycomb.at[base_blk + k], combs[r], ssem[r])

    def wait_stream(k, r):
        _wait(mycomb.at[base_blk + k], combs[r], ssem[r])

    def gather(r):
        for b in range(2):
            pltpu.async_copy(h_hbm.at[combs[r].at[0].at[b]],
                             hrows[r].at[pl.ds(b * 80, 80)], gsem[r])

    def wait_gather(r):
        for b in range(2):
            _wait(h_hbm.at[combs[r].at[0].at[b]],
                  hrows[r].at[pl.ds(b * 80, 80)], gsem[r])

    def scatter(r):
        for b in range(2):
            pltpu.async_copy(hrows[r].at[pl.ds(b * 80, 80)],
                             u_sp.at[idxcs[r].at[b]], csem[r], add=True)

    def wait_scatter(r):
        for b in range(2):
            _wait(hrows[r].at[pl.ds(b * 80, 80)],
                  u_sp.at[idxcs[r].at[b]], csem[r])

    def scale(r):
        for b in range(2):
            for q in range(5):
                sl = pl.ds(q * 16, 16)
                idxc_v[r, b, sl] = comb_v[r, 1, b, sl]

        def row_fn(rr, _):
            rhi = rr // 80
            rlo = rr - rhi * 80
            exs = plsc.load_gather(
                combs[r],
                [jnp.full((16,), 2, jnp.int32),
                 jnp.full((16,), rhi, jnp.int32),
                 jnp.full((16,), rlo, jnp.int32)])
            ex_f = plsc.bitcast(exs, jnp.float32)
            hrows_v[r, rr, pl.ds(0, 16)] = hrows_v[r, rr, pl.ds(0, 16)] * ex_f
            hrows_v[r, rr, pl.ds(16, 16)] = hrows_v[r, rr, pl.ds(16, 16)] * ex_f
            return 0

        lax.fori_loop(0, EB, row_fn, 0)

    stream(0, 0)
    stream(1, 1)
    wait_stream(0, 0)
    gather(0)

    def body(k, _):
        for cur in range(3):
            nxt = (cur + 1) % 3
            prv = (cur + 2) % 3

            @pl.when(lax.rem(k, 3) == cur)
            def _():
                @pl.when(k >= 2)
                def _():
                    wait_scatter(nxt)

                @pl.when(k + 1 < 625)
                def _():
                    wait_stream(k + 1, nxt)
                    gather(nxt)

                wait_gather(cur)
                scale(cur)
                scatter(cur)

                @pl.when(k + 2 < 625)
                def _():
                    stream(k + 2, prv)
        return 0

    lax.fori_loop(0, 625, body, 0)
    wait_scatter(0)
    wait_scatter(2)

    plsc.subcore_barrier()

    zoff = c * 50000 + s * 3128
    pltpu.sync_copy(z01_hbm.at[0].at[pl.ds(zoff, 3128)], zi_v.at[pl.ds(0, 3128)])
    pltpu.sync_copy(z01_hbm.at[1].at[pl.ds(zoff, 3128)], z2_v.at[pl.ds(0, 3128)])

    def zinv_fn(j, _):
        sl = pl.ds(j * 16, 16)
        zc = zi_v[sl] + z2_v[sl]
        zi_v[sl] = jnp.where(zc > 0.0, 1.0 / jnp.where(zc > 0.0, zc, 1.0), 0.0)
        return 0

    lax.fori_loop(0, 196, zinv_fn, 0)

    def uchunk(kk, _):
        row0 = kk * 136
        pltpu.sync_copy(u_sp.at[pl.ds(s * 3128 + row0, 136)], ub_v)

        def urow(rr, _):
            zr = plsc.load_gather(zi_v, [jnp.full((16,), row0 + rr, jnp.int32)])
            ub_v[rr, pl.ds(0, 16)] = ub_v[rr, pl.ds(0, 16)] * zr
            ub_v[rr, pl.ds(16, 16)] = ub_v[rr, pl.ds(16, 16)] * zr
            return 0

        lax.fori_loop(0, 136, urow, 0)
        pltpu.sync_copy(ub_v, u_hbm.at[c].at[pl.ds(s * 3128 + row0, 136)])
        return 0

    lax.fori_loop(0, 23, uchunk, 0)


def _sc_acc(comb, h, z01):
    zu = jnp.zeros((NPAD, HID), jnp.float32)
    return pl.kernel(
        _sc_acc_body,
        out_type=jax.ShapeDtypeStruct((NC, NPAD, HID), jnp.float32),
        mesh=plsc.VectorSubcoreMesh(**_SC_MESH),
        compiler_params=_SC_PARAMS_NL,
        scratch_types=[
            pltpu.VMEM((3, 3, 2, 80), jnp.int32),
            pltpu.VMEM((3, EB, HID), jnp.float32),
            pltpu.VMEM((3, 2, 80), jnp.int32),
            pltpu.VMEM((3136,), jnp.float32),
            pltpu.VMEM((3136,), jnp.float32),
            pltpu.VMEM((136, HID), jnp.float32),
            pltpu.VMEM_SHARED((NPAD, HID), jnp.float32),
            (pltpu.SemaphoreType.DMA,) * 3,
            (pltpu.SemaphoreType.DMA,) * 3,
            (pltpu.SemaphoreType.DMA,) * 3,
        ],
    )(comb, h, zu, z01)


def _node_precompute_kernel(nt_ref, w_ref, b_ref, ht_ref):
    ht_ref[...] = jnp.dot(nt_ref[...], w_ref[...].T,
                          preferred_element_type=jnp.float32) + b_ref[...]


def _edge_precompute_kernel(et_ref, w_ref, b_ref, ae_ref, g_ref, beta_ref,
                            lnet_ref, se_ref):
    et = jnp.dot(et_ref[...], w_ref[...].T,
                 preferred_element_type=jnp.float32) + b_ref[...]
    se_ref[...] = et @ ae_ref[...]
    mu = jnp.mean(et, axis=-1, keepdims=True)
    var = jnp.mean((et - mu) ** 2, axis=-1, keepdims=True)
    lnet_ref[...] = (et - mu) / jnp.sqrt(var + 1e-5) * g_ref[...] + beta_ref[...]


def _final_kernel(u_ref, h_ref, g_ref, beta_ref, out_ref):
    x = u_ref[0] + h_ref[...]
    mu = jnp.mean(x, axis=-1, keepdims=True)
    var = jnp.mean((x - mu) ** 2, axis=-1, keepdims=True)
    out_ref[...] = (x - mu) / jnp.sqrt(var + 1e-5) * g_ref[...] + beta_ref[...]


def kernel(node_features, edge_features, edge_index, node_table, edge_table,
           W_w, b_w, W_e, b_e, attn, ln_gamma, ln_beta):
    a = attn.reshape(3 * HID)
    a_i, a_j, a_e = a[:HID], a[HID:2 * HID], a[2 * HID:]

    n_blocks = pl.cdiv(N_NODES, NODE_BLK)
    ht = pl.pallas_call(
        _node_precompute_kernel,
        grid=(n_blocks,),
        in_specs=[
            pl.BlockSpec((NODE_BLK, EMB), lambda i: (i, 0)),
            pl.BlockSpec((HID, EMB), lambda i: (0, 0)),
            pl.BlockSpec((HID,), lambda i: (0,)),
        ],
        out_specs=pl.BlockSpec((NODE_BLK, HID), lambda i: (i, 0)),
        out_shape=jax.ShapeDtypeStruct((N_NODES, HID), jnp.float32),
    )(node_table, W_w, b_w)

    ln_et, se_t = pl.pallas_call(
        _edge_precompute_kernel,
        out_shape=[
            jax.ShapeDtypeStruct((1000, HID), jnp.float32),
            jax.ShapeDtypeStruct((1000,), jnp.float32),
        ],
    )(edge_table, W_e, b_e, a_e, ln_gamma, ln_beta)

    h = _sc_gather_h(ht, node_features)
    edge_out = _sc_gather_eout(ln_et, edge_features)

    si_n, sj_n = pl.pallas_call(
        _sij_kernel,
        grid=(n_blocks,),
        in_specs=[
            pl.BlockSpec((NODE_BLK, HID), lambda i: (i, 0)),
            pl.BlockSpec((HID,), lambda i: (0,)),
            pl.BlockSpec((HID,), lambda i: (0,)),
        ],
        out_specs=[
            pl.BlockSpec((NODE_BLK,), lambda i: (i,)),
            pl.BlockSpec((NODE_BLK,), lambda i: (i,)),
        ],
        out_shape=[
            jax.ShapeDtypeStruct((N_NODES,), jnp.float32),
            jax.ShapeDtypeStruct((N_NODES,), jnp.float32),
        ],
    )(h, a_i, a_j)

    si_bits = lax.bitcast_convert_type(si_n.astype(jnp.bfloat16), jnp.uint16)
    sj_bits = lax.bitcast_convert_type(sj_n.astype(jnp.bfloat16), jnp.uint16)
    pack = (si_bits.astype(jnp.uint32)
            | (sj_bits.astype(jnp.uint32) << 16)).astype(jnp.int32)

    comb, z01 = _sc_ex(edge_index[0], edge_index[1], edge_features, pack, se_t)
    u_pad = _sc_acc(comb, h, z01)

    out = pl.pallas_call(
        _final_kernel,
        grid=(2, 125),
        in_specs=[
            pl.BlockSpec((1, 400, HID), lambda c, i: (c, i, 0)),
            pl.BlockSpec((400, HID), lambda c, i: (c * 125 + i, 0)),
            pl.BlockSpec((HID,), lambda c, i: (0,)),
            pl.BlockSpec((HID,), lambda c, i: (0,)),
        ],
        out_specs=pl.BlockSpec((400, HID), lambda c, i: (c * 125 + i, 0)),
        out_shape=jax.ShapeDtypeStruct((N_NODES, HID), jnp.float32),
    )(u_pad, h, ln_gamma, ln_beta)

    return (out, edge_out)


# speedup vs baseline: 3.0233x; 3.0233x over previous
import functools

import jax
import jax.numpy as jnp
from jax import lax
from jax.experimental import pallas as pl
from jax.experimental.pallas import tpu as pltpu
from jax.experimental.pallas import tpu_sc as plsc

N_NODES = 100000
N_EDGES = 1600000
EMB = 32
HID = 32
NEG_SLOPE = 0.2

NODE_BLK = 1024

NC = 2
NS = 16
NW = NC * NS

_SC_MESH = dict(core_axis_name="c", subcore_axis_name="s")
_SC_PARAMS = pltpu.CompilerParams(use_tc_tiling_on_sc=False)
_SC_PARAMS_NL = pltpu.CompilerParams(use_tc_tiling_on_sc=False,
                                     needs_layout_passes=False)


def _sc_gather_h_body(ht_hbm, nf_hbm, h_hbm, idx_v, rows_v, sem):
    w = lax.axis_index("s") * NC + lax.axis_index("c")
    nblk = 125
    for k in range(4):
        b = w + k * NW
        @pl.when(b < nblk)
        def _():
            off = b * 800
            pltpu.sync_copy(nf_hbm.at[pl.ds(off, 800)], idx_v)
            pltpu.async_copy(ht_hbm.at[idx_v], rows_v, sem).wait()
            pltpu.sync_copy(rows_v, h_hbm.at[pl.ds(off, 800)])


def _sc_gather_eout_body(lnet_hbm, ef_hbm, eout_hbm, idx_v, rows_a, rows_b, sem):
    w = lax.axis_index("s") * NC + lax.axis_index("c")
    base = w * 50000
    bufs = (rows_a, rows_b)

    pltpu.sync_copy(ef_hbm.at[pl.ds(base, 400)], idx_v.at[0])
    pltpu.async_copy(lnet_hbm.at[idx_v.at[0]], rows_a, sem).wait()

    def step(k, _):
        @pl.when(k + 1 < 125)
        def _():
            pltpu.sync_copy(ef_hbm.at[pl.ds(base + (k + 1) * 400, 400)],
                            idx_v.at[(k + 1) % 2])
        for p in range(2):
            @pl.when(lax.rem(k, 2) == p)
            def _():
                @pl.when(k + 1 < 125)
                def _():
                    cp = pltpu.async_copy(lnet_hbm.at[idx_v.at[(p + 1) % 2]],
                                          bufs[(p + 1) % 2], sem)
                    pltpu.sync_copy(bufs[p], eout_hbm.at[pl.ds(base + k * 400, 400)])
                    cp.wait()
                @pl.when(k + 1 >= 125)
                def _():
                    pltpu.sync_copy(bufs[p], eout_hbm.at[pl.ds(base + k * 400, 400)])
        return 0

    lax.fori_loop(0, 125, step, 0)


def _sc_gather_h(ht, node_features):
    return pl.kernel(
        _sc_gather_h_body,
        out_type=jax.ShapeDtypeStruct((N_NODES, HID), jnp.float32),
        mesh=plsc.VectorSubcoreMesh(**_SC_MESH),
        compiler_params=_SC_PARAMS,
        scratch_types=[
            pltpu.VMEM((800,), jnp.int32),
            pltpu.VMEM((800, HID), jnp.float32),
            pltpu.SemaphoreType.DMA,
        ],
    )(ht, node_features)


def _sc_gather_eout(ln_et, edge_features):
    return pl.kernel(
        _sc_gather_eout_body,
        out_type=jax.ShapeDtypeStruct((N_EDGES, HID), jnp.float32),
        mesh=plsc.VectorSubcoreMesh(**_SC_MESH),
        compiler_params=_SC_PARAMS,
        scratch_types=[
            pltpu.VMEM((2, 400), jnp.int32),
            pltpu.VMEM((400, HID), jnp.float32),
            pltpu.VMEM((400, HID), jnp.float32),
            pltpu.SemaphoreType.DMA,
        ],
    )(ln_et, edge_features)


def _sij_kernel(h_ref, ai_ref, aj_ref, si_ref, sj_ref):
    h = h_ref[...]
    si_ref[...] = h @ ai_ref[...]
    sj_ref[...] = h @ aj_ref[...]


NPAD = 50048
NPADZ = 100096
DUMMY_ROW = 50040
EB = 160
NGB = N_EDGES // EB


def _wait(src, dst, sem):
    pltpu.make_async_copy(src, dst, sem).wait()


def _sc_ex_body(t_hbm, nb_hbm, ef_hbm, pack_hbm, se_hbm, zz_hbm,
                comb_hbm, z01_hbm,
                pack_v, se_v, t_v, nb_v, ef_v, comb0_v, comb1_v, t2_v, ex2_v,
                z_sp, ssem, wsem, zsem):
    c = lax.axis_index("c")
    s = lax.axis_index("s")
    w = s * NC + c
    pltpu.sync_copy(pack_hbm, pack_v)
    pltpu.sync_copy(se_hbm, se_v)
    pltpu.sync_copy(zz_hbm.at[pl.ds(s * 6256, 6256)], z_sp.at[pl.ds(s * 6256, 6256)])
    plsc.subcore_barrier()

    tb = (t_v.at[0], t_v.at[1])
    nbb = (nb_v.at[0], nb_v.at[1])
    efb = (ef_v.at[0], ef_v.at[1])
    c0b = (comb0_v.at[0], comb0_v.at[1])
    c1b = (comb1_v.at[0], comb1_v.at[1])
    t2b = (t2_v.at[0], t2_v.at[1])
    ex2b = (ex2_v.at[0], ex2_v.at[1])

    nblk = jnp.where(w < 16, 313, 312)

    def row_of(k):
        return w + k * NW

    def streams(k, p):
        off = row_of(k) * EB
        pltpu.async_copy(t_hbm.at[pl.ds(off, EB)], tb[p], ssem[p])
        pltpu.async_copy(nb_hbm.at[pl.ds(off, EB)], nbb[p], ssem[p])
        pltpu.async_copy(ef_hbm.at[pl.ds(off, EB)], efb[p], ssem[p])

    def wait_streams(k, p):
        off = row_of(k) * EB
        _wait(t_hbm.at[pl.ds(off, EB)], tb[p], ssem[p])
        _wait(nb_hbm.at[pl.ds(off, EB)], nbb[p], ssem[p])
        _wait(ef_hbm.at[pl.ds(off, EB)], efb[p], ssem[p])

    def wait_outs(k, p):
        gb = row_of(k)
        _wait(c0b[p], comb_hbm.at[0].at[gb], wsem[p])
        _wait(c1b[p], comb_hbm.at[1].at[gb], wsem[p])
        for b in range(2):
            _wait(ex2b[p].at[b], z_sp.at[t2b[p].at[b]], zsem[p])

    streams(0, 0)

    def body(k, _):
        for p in range(2):
            @pl.when((lax.rem(k, 2) == p) & (k < nblk))
            def _():
                @pl.when(k >= 2)
                def _():
                    wait_outs(k, p)

                wait_streams(k, p)

                def chunk(j, _):
                    t16 = tb[p][pl.ds(j * 16, 16)]
                    n16 = nbb[p][pl.ds(j * 16, 16)]
                    ef16 = efb[p][pl.ds(j * 16, 16)]
                    pk_t = plsc.load_gather(pack_v, [t16])
                    pk_n = plsc.load_gather(pack_v, [n16])
                    se16 = plsc.load_gather(se_v, [ef16])
                    si = plsc.bitcast(lax.shift_left(pk_t, 16), jnp.float32)
                    sj = plsc.bitcast(lax.bitwise_and(pk_n, jnp.int32(-65536)),
                                      jnp.float32)
                    x = si + sj + se16
                    x = jnp.where(x >= 0.0, x, NEG_SLOPE * x)
                    ex = jnp.exp(x)
                    owned0 = t16 < 50000
                    exm0 = jnp.where(owned0, ex, 0.0)
                    exm1 = ex - exm0
                    idx0 = jnp.where(owned0, t16, DUMMY_ROW)
                    idx1 = jnp.where(owned0, DUMMY_ROW, t16 - 50000)
                    r = j // 5
                    sl = pl.ds((j % 5) * 16, 16)
                    c0b[p][0, r, sl] = n16
                    c0b[p][1, r, sl] = idx0
                    c0b[p][2, r, sl] = plsc.bitcast(exm0, jnp.int32)
                    c1b[p][0, r, sl] = n16
                    c1b[p][1, r, sl] = idx1
                    c1b[p][2, r, sl] = plsc.bitcast(exm1, jnp.int32)
                    t2b[p][r, sl] = t16
                    ex2b[p][r, sl] = ex
                    return 0

                lax.fori_loop(0, EB // 16, chunk, 0)
                gb = row_of(k)
                pltpu.async_copy(c0b[p], comb_hbm.at[0].at[gb], wsem[p])
                pltpu.async_copy(c1b[p], comb_hbm.at[1].at[gb], wsem[p])
                for b in range(2):
                    pltpu.async_copy(ex2b[p].at[b], z_sp.at[t2b[p].at[b]],
                                     zsem[p], add=True)

                @pl.when(k + 1 < nblk)
                def _():
                    streams(k + 1, 1 - p)
        return 0

    lax.fori_loop(0, 313, body, 0)
    wait_outs(0, 0)
    wait_outs(0, 1)
    plsc.subcore_barrier()
    pltpu.sync_copy(z_sp.at[pl.ds(s * 6256, 6256)],
                    z01_hbm.at[c].at[pl.ds(s * 6256, 6256)])


def _sc_ex(t_arr, nb_arr, ef_arr, pack, se_t):
    zz = jnp.zeros((NPADZ,), jnp.float32)
    return pl.kernel(
        _sc_ex_body,
        out_type=[
            jax.ShapeDtypeStruct((2, NGB, 3, 2, 80), jnp.int32),
            jax.ShapeDtypeStruct((NC, NPADZ), jnp.float32),
        ],
        mesh=plsc.VectorSubcoreMesh(**_SC_MESH),
        compiler_params=_SC_PARAMS_NL,
        scratch_types=[
            pltpu.VMEM((N_NODES,), jnp.int32),
            pltpu.VMEM((1000,), jnp.float32),
            pltpu.VMEM((2, EB), jnp.int32),
            pltpu.VMEM((2, EB), jnp.int32),
            pltpu.VMEM((2, EB), jnp.int32),
            pltpu.VMEM((2, 3, 2, 80), jnp.int32),
            pltpu.VMEM((2, 3, 2, 80), jnp.int32),
            pltpu.VMEM((2, 2, 80), jnp.int32),
            pltpu.VMEM((2, 2, 80), jnp.float32),
            pltpu.VMEM_SHARED((NPADZ,), jnp.float32),
            (pltpu.SemaphoreType.DMA, pltpu.SemaphoreType.DMA),
            (pltpu.SemaphoreType.DMA, pltpu.SemaphoreType.DMA),
            (pltpu.SemaphoreType.DMA, pltpu.SemaphoreType.DMA),
        ],
    )(t_arr, nb_arr, ef_arr, pack, se_t, zz)


def _sc_acc_body(comb_hbm, h_hbm, zu_hbm, z01_hbm, u_hbm,
                 comb_v, hrows_v, idxc_v, zi_v, z2_v, ub_v,
                 u_sp, gsem, csem, ssem):
    c = lax.axis_index("c")
    s = lax.axis_index("s")

    pltpu.sync_copy(zu_hbm.at[pl.ds(s * 3128, 3128)], u_sp.at[pl.ds(s * 3128, 3128)])
    plsc.subcore_barrier()

    mycomb = comb_hbm.at[c]
    base_blk = s * 625
    combs = (comb_v.at[0], comb_v.at[1], comb_v.at[2])
    hrows = (hrows_v.at[0], hrows_v.at[1], hrows_v.at[2])
    idxcs = (idxc_v.at[0], idxc_v.at[1], idxc_v.at[2])

    def stream(k, r):
        pltpu.async_copy(mycomb.at[base_blk + k], combs[r], ssem[r])

    def wait_stream(k, r):
        _wait(mycomb.at[base_blk + k], combs[r], ssem[r])

    def gather(r):
        for b in range(2):
            pltpu.async_copy(h_hbm.at[combs[r].at[0].at[b]],
                             hrows[r].at[pl.ds(b * 80, 80)], gsem[r])

    def wait_gather(r):
        for b in range(2):
            _wait(h_hbm.at[combs[r].at[0].at[b]],
                  hrows[r].at[pl.ds(b * 80, 80)], gsem[r])

    def scatter(r):
        for b in range(2):
            pltpu.async_copy(hrows[r].at[pl.ds(b * 80, 80)],
                             u_sp.at[idxcs[r].at[b]], csem[r], add=True)

    def wait_scatter(r):
        for b in range(2):
            _wait(hrows[r].at[pl.ds(b * 80, 80)],
                  u_sp.at[idxcs[r].at[b]], csem[r])

    def scale(r):
        for b in range(2):
            for q in range(5):
                sl = pl.ds(q * 16, 16)
                idxc_v[r, b, sl] = comb_v[r, 1, b, sl]

        def row_fn(rr, _):
            rhi = rr // 80
            rlo = rr - rhi * 80
            exs = plsc.load_gather(
                combs[r],
                [jnp.full((16,), 2, jnp.int32),
                 jnp.full((16,), rhi, jnp.int32),
                 jnp.full((16,), rlo, jnp.int32)])
            ex_f = plsc.bitcast(exs, jnp.float32)
            hrows_v[r, rr, pl.ds(0, 16)] = hrows_v[r, rr, pl.ds(0, 16)] * ex_f
            hrows_v[r, rr, pl.ds(16, 16)] = hrows_v[r, rr, pl.ds(16, 16)] * ex_f
            return 0

        lax.fori_loop(0, EB, row_fn, 0)

    stream(0, 0)
    stream(1, 1)
    wait_stream(0, 0)
    gather(0)

    def body(k, _):
        for cur in range(3):
            nxt = (cur + 1) % 3
            prv = (cur + 2) % 3

            @pl.when(lax.rem(k, 3) == cur)
            def _():
                @pl.when(k >= 2)
                def _():
                    wait_scatter(nxt)

                @pl.when(k + 1 < 625)
                def _():
                    wait_stream(k + 1, nxt)
                    gather(nxt)

                wait_gather(cur)
                scale(cur)
                scatter(cur)

                @pl.when(k + 2 < 625)
                def _():
                    stream(k + 2, prv)
        return 0

    lax.fori_loop(0, 625, body, 0)
    wait_scatter(0)
    wait_scatter(2)

    plsc.subcore_barrier()

    zoff = c * 50000 + s * 3128
    pltpu.sync_copy(z01_hbm.at[0].at[pl.ds(zoff, 3128)], zi_v.at[pl.ds(0, 3128)])
    pltpu.sync_copy(z01_hbm.at[1].at[pl.ds(zoff, 3128)], z2_v.at[pl.ds(0, 3128)])

    def zinv_fn(j, _):
        sl = pl.ds(j * 16, 16)
        zc = zi_v[sl] + z2_v[sl]
        zi_v[sl] = jnp.where(zc > 0.0, 1.0 / jnp.where(zc > 0.0, zc, 1.0), 0.0)
        return 0

    lax.fori_loop(0, 196, zinv_fn, 0)

    def uchunk(kk, _):
        row0 = kk * 136
        pltpu.sync_copy(u_sp.at[pl.ds(s * 3128 + row0, 136)], ub_v)

        def urow(rr, _):
            zr = plsc.load_gather(zi_v, [jnp.full((16,), row0 + rr, jnp.int32)])
            ub_v[rr, pl.ds(0, 16)] = ub_v[rr, pl.ds(0, 16)] * zr
            ub_v[rr, pl.ds(16, 16)] = ub_v[rr, pl.ds(16, 16)] * zr
            return 0

        lax.fori_loop(0, 136, urow, 0)
        pltpu.sync_copy(ub_v, u_hbm.at[c].at[pl.ds(s * 3128 + row0, 136)])
        return 0

    lax.fori_loop(0, 23, uchunk, 0)


def _sc_acc(comb, h, z01):
    zu = jnp.zeros((NPAD, HID), jnp.float32)
    return pl.kernel(
        _sc_acc_body,
        out_type=jax.ShapeDtypeStruct((NC, NPAD, HID), jnp.float32),
        mesh=plsc.VectorSubcoreMesh(**_SC_MESH),
        compiler_params=_SC_PARAMS_NL,
        scratch_types=[
            pltpu.VMEM((3, 3, 2, 80), jnp.int32),
            pltpu.VMEM((3, EB, HID), jnp.float32),
            pltpu.VMEM((3, 2, 80), jnp.int32),
            pltpu.VMEM((3136,), jnp.float32),
            pltpu.VMEM((3136,), jnp.float32),
            pltpu.VMEM((136, HID), jnp.float32),
            pltpu.VMEM_SHARED((NPAD, HID), jnp.float32),
            (pltpu.SemaphoreType.DMA,) * 3,
            (pltpu.SemaphoreType.DMA,) * 3,
            (pltpu.SemaphoreType.DMA,) * 3,
        ],
    )(comb, h, zu, z01)


def _node_precompute_kernel(nt_ref, w_ref, b_ref, ht_ref):
    ht_ref[...] = jnp.dot(nt_ref[...], w_ref[...].T,
                          preferred_element_type=jnp.float32) + b_ref[...]


def _edge_precompute_kernel(et_ref, w_ref, b_ref, ae_ref, g_ref, beta_ref,
                            lnet_ref, se_ref):
    et = jnp.dot(et_ref[...], w_ref[...].T,
                 preferred_element_type=jnp.float32) + b_ref[...]
    se_ref[...] = et @ ae_ref[...]
    mu = jnp.mean(et, axis=-1, keepdims=True)
    var = jnp.mean((et - mu) ** 2, axis=-1, keepdims=True)
    lnet_ref[...] = (et - mu) / jnp.sqrt(var + 1e-5) * g_ref[...] + beta_ref[...]


def _final_kernel(u_ref, h_ref, g_ref, beta_ref, out_ref):
    x = u_ref[0] + h_ref[...]
    mu = jnp.mean(x, axis=-1, keepdims=True)
    var = jnp.mean((x - mu) ** 2, axis=-1, keepdims=True)
    out_ref[...] = (x - mu) / jnp.sqrt(var + 1e-5) * g_ref[...] + beta_ref[...]


def kernel(node_features, edge_features, edge_index, node_table, edge_table,
           W_w, b_w, W_e, b_e, attn, ln_gamma, ln_beta):
    a = attn.reshape(3 * HID)
    a_i, a_j, a_e = a[:HID], a[HID:2 * HID], a[2 * HID:]

    n_blocks = pl.cdiv(N_NODES, NODE_BLK)
    ht = pl.pallas_call(
        _node_precompute_kernel,
        grid=(n_blocks,),
        in_specs=[
            pl.BlockSpec((NODE_BLK, EMB), lambda i: (i, 0)),
            pl.BlockSpec((HID, EMB), lambda i: (0, 0)),
            pl.BlockSpec((HID,), lambda i: (0,)),
        ],
        out_specs=pl.BlockSpec((NODE_BLK, HID), lambda i: (i, 0)),
        out_shape=jax.ShapeDtypeStruct((N_NODES, HID), jnp.float32),
    )(node_table, W_w, b_w)

    ln_et, se_t = pl.pallas_call(
        _edge_precompute_kernel,
        out_shape=[
            jax.ShapeDtypeStruct((1000, HID), jnp.float32),
            jax.ShapeDtypeStruct((1000,), jnp.float32),
        ],
    )(edge_table, W_e, b_e, a_e, ln_gamma, ln_beta)

    h = _sc_gather_h(ht, node_features)
    edge_out = _sc_gather_eout(ln_et, edge_features)

    si_n, sj_n = pl.pallas_call(
        _sij_kernel,
        grid=(n_blocks,),
        in_specs=[
            pl.BlockSpec((NODE_BLK, HID), lambda i: (i, 0)),
            pl.BlockSpec((HID,), lambda i: (0,)),
            pl.BlockSpec((HID,), lambda i: (0,)),
        ],
        out_specs=[
            pl.BlockSpec((NODE_BLK,), lambda i: (i,)),
            pl.BlockSpec((NODE_BLK,), lambda i: (i,)),
        ],
        out_shape=[
            jax.ShapeDtypeStruct((N_NODES,), jnp.float32),
            jax.ShapeDtypeStruct((N_NODES,), jnp.float32),
        ],
    )(h, a_i, a_j)

    si_bits = lax.bitcast_convert_type(si_n.astype(jnp.bfloat16), jnp.uint16)
    sj_bits = lax.bitcast_convert_type(sj_n.astype(jnp.bfloat16), jnp.uint16)
    pack = (si_bits.astype(jnp.uint32)
            | (sj_bits.astype(jnp.uint32) << 16)).astype(jnp.int32)

    comb, z01 = _sc_ex(edge_index[0], edge_index[1], edge_features, pack, se_t)
    u_pad = _sc_acc(comb, h, z01)

    out = pl.pallas_call(
        _final_kernel,
        grid=(2, 125),
        in_specs=[
            pl.BlockSpec((1, 400, HID), lambda c, i: (c, i, 0)),
            pl.BlockSpec((400, HID), lambda c, i: (c * 125 + i, 0)),
            pl.BlockSpec((HID,), lambda c, i: (0,)),
            pl.BlockSpec((HID,), lambda c, i: (0,)),
        ],
        out_specs=pl.BlockSpec((400, HID), lambda c, i: (c * 125 + i, 0)),
        out_shape=jax.ShapeDtypeStruct((N_NODES, HID), jnp.float32),
    )(u_pad, h, ln_gamma, ln_beta)

    return (out, edge_out)
